# baseline (speedup 1.0000x reference)
import functools

import jax
import jax.numpy as jnp
from jax import lax
from jax.experimental import pallas as pl
from jax.experimental.pallas import tpu as pltpu
from jax.experimental.pallas import tpu_sc as plsc

_NC = 2
_NS = 16
_NW = _NC * _NS
_L = 16
_TBL = 128
_C = 32
_NB = 6
_LEAD = 4


@functools.lru_cache(maxsize=None)
def _make_sc_normalize(n: int, d: int, num_bins: int):
    per_w = n // _NW
    G = per_w // _C
    vregs_per_row = d // _L
    mesh = plsc.VectorSubcoreMesh(core_axis_name="c", subcore_axis_name="s")

    @functools.partial(
        pl.kernel,
        out_type=jax.ShapeDtypeStruct((n, d), jnp.float32),
        mesh=mesh,
        scratch_types=[
            pltpu.VMEM((per_w,), jnp.float32),
            pltpu.VMEM((_TBL,), jnp.float32),
            pltpu.VMEM((_TBL,), jnp.float32),
            pltpu.VMEM((per_w,), jnp.float32),
            pltpu.VMEM((per_w,), jnp.float32),
            *[pltpu.VMEM((_C, d), jnp.float32) for _ in range(_NB)],
            *[pltpu.SemaphoreType.DMA for _ in range(2 * _NB)],
        ],
        compiler_params=pltpu.CompilerParams(needs_layout_passes=False),
    )
    def sc_fn(x_hbm, t_hbm, mean_hbm, std_hbm, out_hbm,
              t_v, mean_v, std_v, scale_v, bias_v, *bufs_and_sems):
        wid = lax.axis_index("s") * _NC + lax.axis_index("c")
        r0 = wid * per_w
        bufs = bufs_and_sems[:_NB]
        in_sems = bufs_and_sems[_NB:2 * _NB]
        out_sems = bufs_and_sems[2 * _NB:]

        in_h = {}
        out_h = {}
        for g in range(min(_LEAD, G)):
            in_h[g] = pltpu.async_copy(
                x_hbm.at[pl.ds(r0 + g * _C, _C), :], bufs[g % _NB], in_sems[g % _NB])

        pltpu.sync_copy(t_hbm.at[pl.ds(r0, per_w)], t_v)
        pltpu.sync_copy(mean_hbm, mean_v)
        pltpu.sync_copy(std_hbm, std_v)

        def gather_body(i, carry):
            sl = pl.ds(i * _L, _L)
            tv = t_v[sl]
            bins = (tv * float(num_bins)).astype(jnp.int32)
            m = plsc.load_gather(mean_v, [bins])
            s = plsc.load_gather(std_v, [bins])
            inv = 1.0 / s
            scale_v[sl] = inv
            bias_v[sl] = -m * inv
            return carry

        lax.fori_loop(0, per_w // _L, gather_body, 0)

        def row_body(buf, g):
            def body(r, carry):
                rr = g * _C + r
                idx = jnp.full((_L,), rr, jnp.int32)
                sv = plsc.load_gather(scale_v, [idx])
                bv = plsc.load_gather(bias_v, [idx])
                for k in range(vregs_per_row):
                    sl = pl.ds(k * _L, _L)
                    buf[r, sl] = buf[r, sl] * sv + bv
                return carry
            lax.fori_loop(0, _C, body, 0)

        waited = set()
        for g in range(G):
            b = g % _NB
            p = g + _LEAD
            if p < G:
                if p - _NB >= 0:
                    out_h[p - _NB].wait()
                    waited.add(p - _NB)
                in_h[p] = pltpu.async_copy(
                    x_hbm.at[pl.ds(r0 + p * _C, _C), :],
                    bufs[p % _NB], in_sems[p % _NB])
            in_h[g].wait()
            row_body(bufs[b], g)
            out_h[g] = pltpu.async_copy(
                bufs[b], out_hbm.at[pl.ds(r0 + g * _C, _C), :], out_sems[b])
        for g in range(G):
            if g not in waited:
                out_h[g].wait()

    return sc_fn


def kernel(x_t, t, data_mean, data_std):
    n, d = x_t.shape
    num_bins = data_mean.shape[0]
    mean_p = jnp.pad(data_mean, (0, _TBL - num_bins))
    std_p = jnp.pad(data_std, (0, _TBL - num_bins), constant_values=1.0)
    return _make_sc_normalize(n, d, num_bins)(x_t, t, mean_p, std_p)


# speedup vs baseline: 1.0261x; 1.0261x over previous
import functools

import jax
import jax.numpy as jnp
from jax import lax
from jax.experimental import pallas as pl
from jax.experimental.pallas import tpu as pltpu
from jax.experimental.pallas import tpu_sc as plsc

_NC = 2
_NS = 16
_NW = _NC * _NS
_L = 16
_TBL = 128
_C = 64


@functools.lru_cache(maxsize=None)
def _make_sc_normalize(n: int, d: int, num_bins: int):
    per_w = n // _NW
    G = per_w // _C
    vregs_per_row = d // _L
    mesh = plsc.VectorSubcoreMesh(core_axis_name="c", subcore_axis_name="s")

    @functools.partial(
        pl.kernel,
        out_type=jax.ShapeDtypeStruct((n, d), jnp.float32),
        mesh=mesh,
        scratch_types=[
            pltpu.VMEM((per_w,), jnp.float32),
            pltpu.VMEM((_TBL,), jnp.float32),
            pltpu.VMEM((_TBL,), jnp.float32),
            pltpu.VMEM((per_w,), jnp.float32),
            pltpu.VMEM((per_w,), jnp.float32),
            pltpu.VMEM((_C, d), jnp.float32),
            pltpu.VMEM((_C, d), jnp.float32),
            pltpu.VMEM((_C, d), jnp.float32),
            pltpu.SemaphoreType.DMA,
            pltpu.SemaphoreType.DMA,
            pltpu.SemaphoreType.DMA,
            pltpu.SemaphoreType.DMA,
            pltpu.SemaphoreType.DMA,
            pltpu.SemaphoreType.DMA,
        ],
        compiler_params=pltpu.CompilerParams(needs_layout_passes=False),
    )
    def sc_fn(x_hbm, t_hbm, mean_hbm, std_hbm, out_hbm,
              t_v, mean_v, std_v, scale_v, bias_v,
              buf0, buf1, buf2, si0, si1, si2, so0, so1, so2):
        wid = lax.axis_index("s") * _NC + lax.axis_index("c")
        r0 = wid * per_w
        bufs = (buf0, buf1, buf2)
        in_sems = (si0, si1, si2)
        out_sems = (so0, so1, so2)

        in_h = {}
        out_h = {}
        for g in range(min(2, G)):
            in_h[g] = pltpu.async_copy(
                x_hbm.at[pl.ds(r0 + g * _C, _C), :], bufs[g % 3], in_sems[g % 3])

        h_t = pltpu.async_copy(t_hbm.at[pl.ds(r0, per_w)], t_v, out_sems[0])
        h_m = pltpu.async_copy(mean_hbm, mean_v, out_sems[1])
        h_s = pltpu.async_copy(std_hbm, std_v, out_sems[2])
        h_t.wait()
        h_m.wait()
        h_s.wait()

        def gather_body(i, carry):
            sl = pl.ds(i * _L, _L)
            tv = t_v[sl]
            bins = (tv * float(num_bins)).astype(jnp.int32)
            m = plsc.load_gather(mean_v, [bins])
            s = plsc.load_gather(std_v, [bins])
            inv = 1.0 / s
            scale_v[sl] = inv
            bias_v[sl] = -m * inv
            return carry

        lax.fori_loop(0, per_w // _L, gather_body, 0)

        def row_body(buf, g):
            def body(r, carry):
                rr = g * _C + r
                idx = jnp.full((_L,), rr, jnp.int32)
                sv = plsc.load_gather(scale_v, [idx])
                bv = plsc.load_gather(bias_v, [idx])
                for k in range(vregs_per_row):
                    sl = pl.ds(k * _L, _L)
                    buf[r, sl] = buf[r, sl] * sv + bv
                return carry
            lax.fori_loop(0, _C, body, 0)

        waited = set()
        for g in range(G):
            b = g % 3
            p = g + 2
            if p < G:
                if p - 3 >= 0:
                    out_h[p - 3].wait()
                    waited.add(p - 3)
                in_h[p] = pltpu.async_copy(
                    x_hbm.at[pl.ds(r0 + p * _C, _C), :], bufs[p % 3], in_sems[p % 3])
            in_h[g].wait()
            row_body(bufs[b], g)
            out_h[g] = pltpu.async_copy(
                bufs[b], out_hbm.at[pl.ds(r0 + g * _C, _C), :], out_sems[b])
        for g in range(G):
            if g not in waited:
                out_h[g].wait()

    return sc_fn


def kernel(x_t, t, data_mean, data_std):
    n, d = x_t.shape
    num_bins = data_mean.shape[0]
    mean_p = jnp.pad(data_mean, (0, _TBL - num_bins))
    std_p = jnp.pad(data_std, (0, _TBL - num_bins), constant_values=1.0)
    return _make_sc_normalize(n, d, num_bins)(x_t, t, mean_p, std_p)


# speedup vs baseline: 1.0274x; 1.0013x over previous
import functools

import jax
import jax.numpy as jnp
from jax import lax
from jax.experimental import pallas as pl
from jax.experimental.pallas import tpu as pltpu
from jax.experimental.pallas import tpu_sc as plsc

_NC = 2
_NS = 16
_NW = _NC * _NS
_L = 16
_TBL = 128
_C = 64


@functools.lru_cache(maxsize=None)
def _make_sc_normalize(n: int, d: int, num_bins: int):
    per_w = n // _NW
    G = per_w // _C
    vregs_per_row = d // _L
    mesh = plsc.VectorSubcoreMesh(core_axis_name="c", subcore_axis_name="s")

    @functools.partial(
        pl.kernel,
        out_type=jax.ShapeDtypeStruct((n, d), jnp.float32),
        mesh=mesh,
        scratch_types=[
            pltpu.VMEM((per_w,), jnp.float32),
            pltpu.VMEM((_TBL,), jnp.float32),
            pltpu.VMEM((_TBL,), jnp.float32),
            pltpu.VMEM((per_w,), jnp.float32),
            pltpu.VMEM((per_w,), jnp.float32),
            pltpu.VMEM((_C, d), jnp.float32),
            pltpu.VMEM((_C, d), jnp.float32),
            pltpu.VMEM((_C, d), jnp.float32),
            pltpu.SemaphoreType.DMA,
            pltpu.SemaphoreType.DMA,
            pltpu.SemaphoreType.DMA,
            pltpu.SemaphoreType.DMA,
            pltpu.SemaphoreType.DMA,
            pltpu.SemaphoreType.DMA,
        ],
        compiler_params=pltpu.CompilerParams(
            needs_layout_passes=False,
            skip_device_barrier=True,
            disable_bounds_checks=True,
        ),
    )
    def sc_fn(x_hbm, t_hbm, mean_hbm, std_hbm, out_hbm,
              t_v, mean_v, std_v, scale_v, bias_v,
              buf0, buf1, buf2, si0, si1, si2, so0, so1, so2):
        wid = lax.axis_index("s") * _NC + lax.axis_index("c")
        r0 = wid * per_w
        bufs = (buf0, buf1, buf2)
        in_sems = (si0, si1, si2)
        out_sems = (so0, so1, so2)

        in_h = {}
        out_h = {}
        for g in range(min(2, G)):
            in_h[g] = pltpu.async_copy(
                x_hbm.at[pl.ds(r0 + g * _C, _C), :], bufs[g % 3], in_sems[g % 3])

        h_t = pltpu.async_copy(t_hbm.at[pl.ds(r0, per_w)], t_v, out_sems[0])
        h_m = pltpu.async_copy(mean_hbm, mean_v, out_sems[1])
        h_s = pltpu.async_copy(std_hbm, std_v, out_sems[2])
        h_t.wait()
        h_m.wait()
        h_s.wait()

        def gather_body(i, carry):
            sl = pl.ds(i * _L, _L)
            tv = t_v[sl]
            bins = (tv * float(num_bins)).astype(jnp.int32)
            m = plsc.load_gather(mean_v, [bins])
            s = plsc.load_gather(std_v, [bins])
            inv = 1.0 / s
            scale_v[sl] = inv
            bias_v[sl] = -m * inv
            return carry

        lax.fori_loop(0, per_w // _L, gather_body, 0)

        def row_body(buf, g):
            def body(r, carry):
                rr = g * _C + r
                idx = jnp.full((_L,), rr, jnp.int32)
                sv = plsc.load_gather(scale_v, [idx])
                bv = plsc.load_gather(bias_v, [idx])
                for k in range(vregs_per_row):
                    sl = pl.ds(k * _L, _L)
                    buf[r, sl] = buf[r, sl] * sv + bv
                return carry
            lax.fori_loop(0, _C, body, 0)

        waited = set()
        for g in range(G):
            b = g % 3
            p = g + 2
            if p < G:
                if p - 3 >= 0:
                    out_h[p - 3].wait()
                    waited.add(p - 3)
                in_h[p] = pltpu.async_copy(
                    x_hbm.at[pl.ds(r0 + p * _C, _C), :], bufs[p % 3], in_sems[p % 3])
            in_h[g].wait()
            row_body(bufs[b], g)
            out_h[g] = pltpu.async_copy(
                bufs[b], out_hbm.at[pl.ds(r0 + g * _C, _C), :], out_sems[b])
        for g in range(G):
            if g not in waited:
                out_h[g].wait()

    return sc_fn


def kernel(x_t, t, data_mean, data_std):
    n, d = x_t.shape
    num_bins = data_mean.shape[0]
    mean_p = jnp.pad(data_mean, (0, _TBL - num_bins))
    std_p = jnp.pad(data_std, (0, _TBL - num_bins), constant_values=1.0)
    return _make_sc_normalize(n, d, num_bins)(x_t, t, mean_p, std_p)
